# Initial kernel scaffold; baseline (speedup 1.0000x reference)
import jax
import jax.numpy as jnp
from jax.experimental import pallas as pl


def kernel(x, h, Wk, Wv, Wq):
    raise NotImplementedError("write your pallas kernel here")


# speedup vs baseline: 1.3303x; 1.3303x over previous
import functools
import math

import jax
import jax.numpy as jnp
from jax.experimental import pallas as pl

S = 64
INPUT = 64
HID = 64
KD = 16
VD = 16
H = 2
N = 16
K = 8
EPS = 1e-08

BB = 256
_PREC = jax.lax.Precision.DEFAULT


def _attn_block(x_ref, h_ref, wkT_ref, wvT_ref, wq_ref, inp_ref, mask_ref, probs_ref):
    bb = x_ref.shape[0] // S
    xf = x_ref[:]
    key = jnp.dot(xf, wkT_ref[:], preferred_element_type=jnp.float32, precision=_PREC)
    val = jnp.dot(xf, wvT_ref[:], preferred_element_type=jnp.float32, precision=_PREC)
    value_m = (0.5 * (val[:, :VD] + val[:, VD:])).reshape(bb, S, VD)
    key = key.reshape(bb, S, H * KD)

    h_all = h_ref[:].reshape(bb, N, HID)
    qs = [jnp.dot(h_all[:, n, :], wq_ref[n], preferred_element_type=jnp.float32,
                  precision=_PREC) for n in range(N)]
    query = jnp.stack(qs, axis=1)

    scale = 1.0 / (H * math.sqrt(KD))
    s1 = jnp.einsum('bnd,bsd->bns', query[:, :, :KD], key[:, :, :KD],
                    precision=_PREC, preferred_element_type=jnp.float32)
    s2 = jnp.einsum('bnd,bsd->bns', query[:, :, KD:], key[:, :, KD:],
                    precision=_PREC, preferred_element_type=jnp.float32)
    scores = (s1 + s2) * scale

    m = jnp.max(scores, axis=1, keepdims=True)
    e = jnp.exp(scores - m)
    probs = e / jnp.sum(e, axis=1, keepdims=True)
    probs = probs + EPS
    probs = probs / jnp.sum(probs, axis=2, keepdims=True)
    probs_ref[:] = probs

    v = 1.0 - probs[:, :, S - 1]
    rank = jnp.zeros((bb, N), dtype=jnp.float32)
    for d in range(1, N):
        w = jnp.roll(v, -d, axis=1)
        tie_lt = jax.lax.broadcasted_iota(jnp.int32, (1, N), 1) >= (N - d)
        beats = (w > v) | ((w == v) & tie_lt)
        rank = rank + beats.astype(jnp.float32)
    mask = (rank < float(K)).astype(jnp.float32)
    mask_ref[:] = mask

    out = jnp.einsum('bns,bsv->bnv', probs, value_m, precision=_PREC,
                     preferred_element_type=jnp.float32)
    inp_ref[:] = out * mask[:, :, None]


@functools.partial(jax.jit, static_argnames=())
def kernel(x, h, Wk, Wv, Wq):
    B = x.shape[0]
    x2 = x.reshape(B * S, INPUT)
    h2 = h.reshape(B, N * HID)

    grid = (B // BB,)
    out = pl.pallas_call(
        _attn_block,
        grid=grid,
        in_specs=[
            pl.BlockSpec((BB * S, INPUT), lambda i: (i, 0)),
            pl.BlockSpec((BB, N * HID), lambda i: (i, 0)),
            pl.BlockSpec((INPUT, H * KD), lambda i: (0, 0)),
            pl.BlockSpec((INPUT, H * VD), lambda i: (0, 0)),
            pl.BlockSpec((N, HID, H * KD), lambda i: (0, 0, 0)),
        ],
        out_specs=[
            pl.BlockSpec((BB, N, VD), lambda i: (i, 0, 0)),
            pl.BlockSpec((BB, N), lambda i: (i, 0)),
            pl.BlockSpec((BB, N, S), lambda i: (i, 0, 0)),
        ],
        out_shape=[
            jax.ShapeDtypeStruct((B, N, VD), jnp.float32),
            jax.ShapeDtypeStruct((B, N), jnp.float32),
            jax.ShapeDtypeStruct((B, N, S), jnp.float32),
        ],
    )(x2, h2, Wk.T, Wv.T, Wq)
    return tuple(out)


# speedup vs baseline: 1.6338x; 1.2281x over previous
import functools
import math

import jax
import jax.numpy as jnp
from jax.experimental import pallas as pl

S = 64
INPUT = 64
HID = 64
KD = 16
VD = 16
H = 2
N = 16
K = 8
EPS = 1e-08

BB = 256
_PREC = jax.lax.Precision.DEFAULT


def _attn_block(x_ref, h_ref, wkT_ref, wvT_ref, wq_ref, inp_ref, mask_ref, probs_ref):
    bb = x_ref.shape[0] // S
    xf = x_ref[:]
    key = jnp.dot(xf, wkT_ref[:], preferred_element_type=jnp.float32, precision=_PREC)
    val = jnp.dot(xf, wvT_ref[:], preferred_element_type=jnp.float32, precision=_PREC)
    value_m = (0.5 * (val[:, :VD] + val[:, VD:])).reshape(bb, S, VD)
    key = key.reshape(bb, S, H * KD)

    qf = jnp.dot(h_ref[:], wq_ref[:], preferred_element_type=jnp.float32,
                 precision=_PREC)
    query = qf.reshape(bb, N, H * KD)

    scale = 1.0 / (H * math.sqrt(KD))
    scores = jnp.einsum('bnd,bsd->bns', query, key, precision=_PREC,
                        preferred_element_type=jnp.float32) * scale

    m = jnp.max(scores, axis=1, keepdims=True)
    e = jnp.exp(scores - m)
    probs = e / jnp.sum(e, axis=1, keepdims=True)
    probs = probs + EPS
    probs = probs / jnp.sum(probs, axis=2, keepdims=True)
    probs_ref[:] = probs

    v = 1.0 - probs[:, :, S - 1]
    rank = jnp.zeros((bb, N), dtype=jnp.float32)
    for d in range(1, N):
        w = jnp.roll(v, -d, axis=1)
        tie_lt = jax.lax.broadcasted_iota(jnp.int32, (1, N), 1) >= (N - d)
        beats = (w > v) | ((w == v) & tie_lt)
        rank = rank + beats.astype(jnp.float32)
    mask = (rank < float(K)).astype(jnp.float32)
    mask_ref[:] = mask

    out = jnp.einsum('bns,bsv->bnv', probs, value_m, precision=_PREC,
                     preferred_element_type=jnp.float32)
    inp_ref[:] = out * mask[:, :, None]


@functools.partial(jax.jit, static_argnames=())
def kernel(x, h, Wk, Wv, Wq):
    B = x.shape[0]
    x2 = x.reshape(B * S, INPUT)
    h2 = h.reshape(B, N * HID)

    wq_bd = jnp.zeros((N, HID, N, H * KD), dtype=Wq.dtype)
    idx = jnp.arange(N)
    wq_bd = wq_bd.at[idx, :, idx, :].set(Wq).reshape(N * HID, N * H * KD)

    grid = (B // BB,)
    out = pl.pallas_call(
        _attn_block,
        grid=grid,
        in_specs=[
            pl.BlockSpec((BB * S, INPUT), lambda i: (i, 0)),
            pl.BlockSpec((BB, N * HID), lambda i: (i, 0)),
            pl.BlockSpec((INPUT, H * KD), lambda i: (0, 0)),
            pl.BlockSpec((INPUT, H * VD), lambda i: (0, 0)),
            pl.BlockSpec((N * HID, N * H * KD), lambda i: (0, 0)),
        ],
        out_specs=[
            pl.BlockSpec((BB, N, VD), lambda i: (i, 0, 0)),
            pl.BlockSpec((BB, N), lambda i: (i, 0)),
            pl.BlockSpec((BB, N, S), lambda i: (i, 0, 0)),
        ],
        out_shape=[
            jax.ShapeDtypeStruct((B, N, VD), jnp.float32),
            jax.ShapeDtypeStruct((B, N), jnp.float32),
            jax.ShapeDtypeStruct((B, N, S), jnp.float32),
        ],
    )(x2, h2, Wk.T, Wv.T, wq_bd)
    return tuple(out)
